# baseline (speedup 1.0000x reference)
import dataclasses
import functools

import jax
import jax.numpy as jnp
from jax import lax
from jax.experimental import pallas as pl
from jax.experimental.pallas import tpu as pltpu
from jax.experimental.pallas import tpu_sc as plsc

NS = 16
CH = 80
BM = 1024


def _mesh():
    return plsc.VectorSubcoreMesh(core_axis_name="c", subcore_axis_name="s")


def _no_layout():
    return dataclasses.replace(pltpu.CompilerParams(),
                               needs_layout_passes=False)


@functools.lru_cache(maxsize=None)
def _deg_call(N: int, E: int):
    e_per_tile = E // (2 * NS)

    @functools.partial(
        pl.kernel,
        out_type=jax.ShapeDtypeStruct((2 * NS, N), jnp.float32),
        mesh=_mesh(),
        scratch_types=[
            pltpu.VMEM((E // (2 * NS),), jnp.int32),
            pltpu.VMEM((N,), jnp.float32),
        ],
        compiler_params=_no_layout(),
    )
    def deg_kernel(dst_hbm, out_hbm, idx_d, hist_v):
        cid = lax.axis_index("c")
        sid = lax.axis_index("s")
        wid = cid * NS + sid
        pltpu.sync_copy(dst_hbm.at[pl.ds(wid * e_per_tile, e_per_tile)],
                        idx_d)

        @pl.loop(0, N, step=16)
        def _(i):
            hist_v[pl.ds(i, 16)] = jnp.zeros((16,), jnp.float32)

        ones_reg = jnp.full((16,), 1.0, jnp.float32)

        @pl.loop(0, e_per_tile, step=80)
        def _(k):
            for j in range(5):
                iv = idx_d[pl.ds(k + j * 16, 16)]
                plsc.addupdate_scatter(hist_v, [iv], ones_reg)

        pltpu.sync_copy(hist_v, out_hbm.at[wid])

    return deg_kernel


@functools.lru_cache(maxsize=None)
def _agg_call(N: int, E: int, H: int):
    e_per_tile = E // NS
    rpt = N // NS
    BCH = 50 * CH

    @functools.partial(
        pl.kernel,
        out_type=(jax.ShapeDtypeStruct((N, H), jnp.float32),
                  jax.ShapeDtypeStruct((N, H), jnp.float32)),
        mesh=_mesh(),
        scratch_types=[
            pltpu.VMEM((BCH,), jnp.int32),
            pltpu.VMEM((BCH,), jnp.int32),
            pltpu.VMEM((CH,), jnp.int32),
            pltpu.VMEM((CH,), jnp.int32),
            pltpu.VMEM((CH,), jnp.int32),
            pltpu.VMEM((CH, H), jnp.float32),
            pltpu.VMEM((CH, H), jnp.float32),
            pltpu.VMEM_SHARED((N, H), jnp.float32),
            pltpu.SemaphoreType.DMA,
            pltpu.SemaphoreType.DMA,
        ],
    )
    def agg_kernel(y2_hbm, s0_hbm, s1_hbm, dst_hbm, z_hbm,
                   outlo_hbm, outhi_hbm, sbatch, dbatch, is0, is1, idx_d,
                   rows0, rows1, acc, sem0, sem1):
        cid = lax.axis_index("c")
        sid = lax.axis_index("s")
        pltpu.sync_copy(z_hbm, acc.at[pl.ds(sid * rpt, rpt)])
        plsc.subcore_barrier()
        tbase = sid * e_per_tile
        nj = BCH // CH
        isb = (is0, is1)
        rowsb = (rows0, rows1)
        semb = (sem0, sem1)

        def repack(dst_v, src_v, j):
            for i in range(CH // 16):
                dst_v[pl.ds(i * 16, 16)] = src_v[pl.ds(j * CH + i * 16, 16)]

        def process(s_hbm, out_hbm):
            @pl.loop(0, e_per_tile, step=BCH)
            def _(kb):
                pltpu.sync_copy(s_hbm.at[pl.ds(tbase + kb, BCH)], sbatch)
                pltpu.sync_copy(dst_hbm.at[pl.ds(tbase + kb, BCH)], dbatch)
                repack(is0, sbatch, 0)
                pltpu.async_copy(y2_hbm.at[is0], rows0, sem0)
                for j in range(nj):
                    cur, nxt = j % 2, (j + 1) % 2
                    if j + 1 < nj:
                        repack(isb[nxt], sbatch, j + 1)
                        pltpu.async_copy(y2_hbm.at[isb[nxt]], rowsb[nxt],
                                         semb[nxt])
                    pltpu.make_async_copy(y2_hbm.at[isb[cur]], rowsb[cur],
                                          semb[cur]).wait()
                    repack(idx_d, dbatch, j)
                    pltpu.sync_copy(rowsb[cur], acc.at[idx_d], add=True)

            plsc.subcore_barrier()
            pltpu.sync_copy(acc.at[pl.ds(sid * rpt, rpt)],
                            out_hbm.at[pl.ds(sid * rpt, rpt)])

        @pl.when(cid == 0)
        def _():
            process(s0_hbm, outlo_hbm)

        @pl.when(cid == 1)
        def _():
            process(s1_hbm, outhi_hbm)

    return agg_kernel


def _mm1_body(x_ref, w_ref, o_ref):
    o_ref[...] = jnp.dot(x_ref[...], w_ref[...],
                         preferred_element_type=jnp.float32)


@functools.lru_cache(maxsize=None)
def _mm1(N, K, C):
    return pl.pallas_call(
        _mm1_body,
        grid=(N // BM,),
        in_specs=[pl.BlockSpec((BM, K), lambda i: (i, 0)),
                  pl.BlockSpec((K, C), lambda i: (0, 0))],
        out_specs=pl.BlockSpec((BM, C), lambda i: (i, 0)),
        out_shape=jax.ShapeDtypeStruct((N, C), jnp.float32),
    )


def _e1_body(dg_ref, xw_ref, dis_ref, y_ref):
    ones32 = jnp.ones((dg_ref.shape[0], 1), jnp.float32)
    cnt = lax.dot_general(dg_ref[...], ones32, (((0,), (0,)), ((), ())),
                          preferred_element_type=jnp.float32)
    dis = lax.rsqrt(1.0 + cnt)
    dis_ref[...] = dis
    y_ref[...] = dis * xw_ref[...]


@functools.lru_cache(maxsize=None)
def _e1(N, C):
    return pl.pallas_call(
        _e1_body,
        grid=(N // BM,),
        in_specs=[pl.BlockSpec((2 * NS, BM), lambda i: (0, i)),
                  pl.BlockSpec((BM, C), lambda i: (i, 0))],
        out_specs=(pl.BlockSpec((BM, 1), lambda i: (i, 0)),
                   pl.BlockSpec((BM, C), lambda i: (i, 0))),
        out_shape=(jax.ShapeDtypeStruct((N, 1), jnp.float32),
                   jax.ShapeDtypeStruct((N, C), jnp.float32)),
    )


def _k2_body(dis_ref, alo_ref, ahi_ref, xw_ref, b_ref, w_ref,
             xw2_ref, y_ref):
    dis = dis_ref[...]
    agg = jnp.concatenate([alo_ref[...], ahi_ref[...]], axis=1)
    h = jnp.maximum(dis * agg + (dis * dis) * xw_ref[...] + b_ref[...], 0.0)
    xw2 = jnp.dot(h, w_ref[...], preferred_element_type=jnp.float32)
    xw2_ref[...] = xw2
    y_ref[...] = dis * xw2


@functools.lru_cache(maxsize=None)
def _k2(N, C, C2):
    H = C // 2
    return pl.pallas_call(
        _k2_body,
        grid=(N // BM,),
        in_specs=[pl.BlockSpec((BM, 1), lambda i: (i, 0)),
                  pl.BlockSpec((BM, H), lambda i: (i, 0)),
                  pl.BlockSpec((BM, H), lambda i: (i, 0)),
                  pl.BlockSpec((BM, C), lambda i: (i, 0)),
                  pl.BlockSpec((1, C), lambda i: (0, 0)),
                  pl.BlockSpec((C, C2), lambda i: (0, 0))],
        out_specs=(pl.BlockSpec((BM, C2), lambda i: (i, 0)),
                   pl.BlockSpec((BM, C2), lambda i: (i, 0))),
        out_shape=(jax.ShapeDtypeStruct((N, C2), jnp.float32),
                   jax.ShapeDtypeStruct((N, C2), jnp.float32)),
    )


def _k3_body(dis_ref, alo_ref, ahi_ref, xw_ref, b_ref, w_ref, blin_ref,
             o_ref):
    dis = dis_ref[...]
    agg = jnp.concatenate([alo_ref[...], ahi_ref[...]], axis=1)
    h = jnp.maximum(dis * agg + (dis * dis) * xw_ref[...] + b_ref[...], 0.0)
    o_ref[...] = jnp.dot(h, w_ref[...],
                         preferred_element_type=jnp.float32) + blin_ref[...]


@functools.lru_cache(maxsize=None)
def _k3(N, C, O):
    H = C // 2
    return pl.pallas_call(
        _k3_body,
        grid=(N // BM,),
        in_specs=[pl.BlockSpec((BM, 1), lambda i: (i, 0)),
                  pl.BlockSpec((BM, H), lambda i: (i, 0)),
                  pl.BlockSpec((BM, H), lambda i: (i, 0)),
                  pl.BlockSpec((BM, C), lambda i: (i, 0)),
                  pl.BlockSpec((1, C), lambda i: (0, 0)),
                  pl.BlockSpec((C, O), lambda i: (0, 0)),
                  pl.BlockSpec((1, O), lambda i: (0, 0))],
        out_specs=pl.BlockSpec((BM, O), lambda i: (i, 0)),
        out_shape=jax.ShapeDtypeStruct((N, O), jnp.float32),
    )


def kernel(x, edge_index, W1, b1, W2, b2, Wlin, blin):
    N, Cin = x.shape
    E = edge_index.shape[1]
    C = W1.shape[1]
    C2 = W2.shape[1]
    O = Wlin.shape[1]
    H = C // 2

    NP = ((N + BM - 1) // BM) * BM

    xp = jnp.pad(x, ((0, NP - N), (0, 0)))
    src = edge_index[0].astype(jnp.int32)
    dst = edge_index[1].astype(jnp.int32)
    s0 = src * 2
    s1 = src * 2 + 1
    z = jnp.zeros((NP // NS, H), jnp.float32)

    degp = _deg_call(NP, E)(dst)
    xw1 = _mm1(NP, Cin, C)(xp, W1)
    dis, y1 = _e1(NP, C)(degp, xw1)
    a1lo, a1hi = _agg_call(NP, E, H)(y1.reshape(2 * NP, H), s0, s1, dst, z)
    xw2, y2 = _k2(NP, C, C2)(dis, a1lo, a1hi, xw1, b1.reshape(1, -1), W2)
    a2lo, a2hi = _agg_call(NP, E, C2 // 2)(y2.reshape(2 * NP, C2 // 2),
                                           s0, s1, dst, z)
    out = _k3(NP, C2, O)(dis, a2lo, a2hi, xw2, b2.reshape(1, -1),
                         Wlin, blin.reshape(1, -1))
    return out[:N]
